# Initial kernel scaffold; baseline (speedup 1.0000x reference)
import jax
import jax.numpy as jnp
from jax.experimental import pallas as pl


def kernel(frame_features, slow_result, fast_result, W_frame, b_frame, W_fast, b_fast, Wt, bt, Wg, bg):
    raise NotImplementedError("write your pallas kernel here")


# speedup vs baseline: 24.9901x; 24.9901x over previous
import math

import jax
import jax.numpy as jnp
from jax.experimental import pallas as pl

_T = 16
_K = 4
_C = 256


def _fused_kernel(x_ref, topic_ref, wf_ref, bf_ref, wfast_ref, bfast_ref,
                  wt_ref, bt_ref, wg_ref, bg_ref, out_ref):
    G = topic_ref.shape[0]
    P = jnp.dot(x_ref[...], wf_ref[...], preferred_element_type=jnp.float32)
    P = P + bf_ref[...]
    fast_pre = jnp.dot(topic_ref[...], wfast_ref[...],
                       preferred_element_type=jnp.float32) + bfast_ref[...]
    t = jnp.tanh(jnp.dot(fast_pre, wt_ref[...],
                         preferred_element_type=jnp.float32) + bt_ref[...])
    gate = jax.nn.sigmoid(t)

    P3 = P.reshape(G, _T, _C)
    xm = P3 * gate[:, None, :]

    sim = jax.lax.dot_general(xm, xm, (((2,), (2,)), ((0,), (0,))),
                              preferred_element_type=jnp.float32)
    sim = sim * (1.0 / math.sqrt(_C))

    lane = jax.lax.broadcasted_iota(jnp.int32, (G, _T, _T), 2)
    s_work = sim
    vals = []
    hots = []
    for _ in range(_K):
        m = jnp.max(s_work, axis=-1, keepdims=True)
        eq = s_work == m
        first = jnp.min(jnp.where(eq, lane, _T), axis=-1, keepdims=True)
        onehot = lane == first
        vals.append(m)
        hots.append(onehot)
        s_work = jnp.where(onehot, -jnp.inf, s_work)
    v0 = vals[0]
    es = [jnp.exp(v - v0) for v in vals]
    denom = es[0] + es[1] + es[2] + es[3]
    A = jnp.zeros_like(sim)
    for j in range(_K):
        A = A + jnp.where(hots[j], es[j] / denom, 0.0)

    agg = jax.lax.dot_general(A, P3, (((2,), (1,)), ((0,), (0,))),
                              preferred_element_type=jnp.float32)

    out = jnp.dot(agg.reshape(G * _T, _C), wg_ref[...],
                  preferred_element_type=jnp.float32) + bg_ref[...]
    out_ref[...] = jnp.maximum(out + P, 0.0)


def kernel(frame_features, slow_result, fast_result, W_frame, b_frame,
           W_fast, b_fast, Wt, bt, Wg, bg):
    B, L, FD = frame_features.shape
    LF = fast_result.shape[1]
    C = W_frame.shape[1]
    target_len = math.ceil(L / 16) * 16
    x2d = frame_features.reshape(B * L, FD)
    if target_len != L:
        pad = jnp.zeros((B * (target_len - L), FD), dtype=x2d.dtype)
        x2d = jnp.concatenate([x2d, pad], axis=0)
    N = B * LF

    fast2d = fast_result.reshape(N, -1)

    g, cg, _ = Wg.shape
    Wbig = (jnp.eye(g, dtype=Wg.dtype)[:, None, :, None]
            * Wg[:, :, None, :]).reshape(g * cg, g * cg)

    GRAPHS_PER_TILE = 32
    n_tiles = N // GRAPHS_PER_TILE
    rows = GRAPHS_PER_TILE * _T

    out = pl.pallas_call(
        _fused_kernel,
        grid=(n_tiles,),
        in_specs=[
            pl.BlockSpec((rows, FD), lambda i: (i, 0)),
            pl.BlockSpec((GRAPHS_PER_TILE, fast2d.shape[1]), lambda i: (i, 0)),
            pl.BlockSpec((FD, C), lambda i: (0, 0)),
            pl.BlockSpec((1, C), lambda i: (0, 0)),
            pl.BlockSpec((W_fast.shape[0], C), lambda i: (0, 0)),
            pl.BlockSpec((1, C), lambda i: (0, 0)),
            pl.BlockSpec((C, C), lambda i: (0, 0)),
            pl.BlockSpec((1, C), lambda i: (0, 0)),
            pl.BlockSpec((C, C), lambda i: (0, 0)),
            pl.BlockSpec((1, C), lambda i: (0, 0)),
        ],
        out_specs=pl.BlockSpec((rows, C), lambda i: (i, 0)),
        out_shape=jax.ShapeDtypeStruct((N * _T, C), jnp.float32),
    )(x2d, fast2d, W_frame, b_frame.reshape(1, C), W_fast,
      b_fast.reshape(1, C), Wt, bt.reshape(1, C), Wbig, bg.reshape(1, C))

    return out.reshape(B, target_len, C)


# speedup vs baseline: 32.1979x; 1.2884x over previous
import math

import jax
import jax.numpy as jnp
from jax.experimental import pallas as pl

_T = 16
_K = 4
_C = 256


def _fused_kernel(x_ref, topic_ref, wf_ref, bf_ref, wfast_ref, bfast_ref,
                  wt_ref, bt_ref, wg_ref, bg_ref, out_ref):
    G = topic_ref.shape[0]
    P = jnp.dot(x_ref[...], wf_ref[...], preferred_element_type=jnp.float32)
    P = P + bf_ref[...]
    fast_pre = jnp.dot(topic_ref[...], wfast_ref[...],
                       preferred_element_type=jnp.float32) + bfast_ref[...]
    t = jnp.tanh(jnp.dot(fast_pre, wt_ref[...],
                         preferred_element_type=jnp.float32) + bt_ref[...])
    gate = jax.nn.sigmoid(t)

    P3 = P.reshape(G, _T, _C)
    xm = P3 * gate[:, None, :]

    sim = jax.lax.dot_general(xm, xm, (((2,), (2,)), ((0,), (0,))),
                              preferred_element_type=jnp.float32)
    sim = sim * (1.0 / math.sqrt(_C))

    R = jnp.swapaxes(sim, 0, 1).reshape(_T, G * _T)

    srow = jax.lax.broadcasted_iota(jnp.int32, (_T, G * _T), 0).astype(jnp.float32)
    pw = jnp.exp2(-srow)
    s_work = R
    m0 = None
    for j in range(_K):
        m = jnp.max(s_work, axis=0, keepdims=True)
        if j == 0:
            m0 = m
        q = jnp.where(s_work == m, pw, 0.0)
        mq = jnp.max(q, axis=0, keepdims=True)
        onehot = q == mq
        s_work = jnp.where(onehot, -jnp.inf, s_work)
    e = jnp.where(s_work == -jnp.inf, jnp.exp(R - m0), 0.0)
    denom = jnp.sum(e, axis=0, keepdims=True)
    AB = (e * (1.0 / denom)).reshape(_T, G, _T)

    agg = jax.lax.dot_general(jnp.swapaxes(AB, 0, 1), P3,
                              (((1,), (1,)), ((0,), (0,))),
                              preferred_element_type=jnp.float32)

    out = jnp.dot(agg.reshape(G * _T, _C), wg_ref[...],
                  preferred_element_type=jnp.float32) + bg_ref[...]
    out_ref[...] = jnp.maximum(out + P, 0.0)


def kernel(frame_features, slow_result, fast_result, W_frame, b_frame,
           W_fast, b_fast, Wt, bt, Wg, bg):
    B, L, FD = frame_features.shape
    LF = fast_result.shape[1]
    C = W_frame.shape[1]
    target_len = math.ceil(L / 16) * 16
    x2d = frame_features.reshape(B * L, FD)
    if target_len != L:
        pad = jnp.zeros((B * (target_len - L), FD), dtype=x2d.dtype)
        x2d = jnp.concatenate([x2d, pad], axis=0)
    N = B * LF

    fast2d = fast_result.reshape(N, -1)

    g, cg, _ = Wg.shape
    Wbig = (jnp.eye(g, dtype=Wg.dtype)[:, None, :, None]
            * Wg[:, :, None, :]).reshape(g * cg, g * cg)

    GRAPHS_PER_TILE = 32
    n_tiles = N // GRAPHS_PER_TILE
    rows = GRAPHS_PER_TILE * _T

    out = pl.pallas_call(
        _fused_kernel,
        grid=(n_tiles,),
        in_specs=[
            pl.BlockSpec((rows, FD), lambda i: (i, 0)),
            pl.BlockSpec((GRAPHS_PER_TILE, fast2d.shape[1]), lambda i: (i, 0)),
            pl.BlockSpec((FD, C), lambda i: (0, 0)),
            pl.BlockSpec((1, C), lambda i: (0, 0)),
            pl.BlockSpec((W_fast.shape[0], C), lambda i: (0, 0)),
            pl.BlockSpec((1, C), lambda i: (0, 0)),
            pl.BlockSpec((C, C), lambda i: (0, 0)),
            pl.BlockSpec((1, C), lambda i: (0, 0)),
            pl.BlockSpec((C, C), lambda i: (0, 0)),
            pl.BlockSpec((1, C), lambda i: (0, 0)),
        ],
        out_specs=pl.BlockSpec((rows, C), lambda i: (i, 0)),
        out_shape=jax.ShapeDtypeStruct((N * _T, C), jnp.float32),
    )(x2d, fast2d, W_frame, b_frame.reshape(1, C), W_fast,
      b_fast.reshape(1, C), Wt, bt.reshape(1, C), Wbig, bg.reshape(1, C))

    return out.reshape(B, target_len, C)


# speedup vs baseline: 40.1744x; 1.2477x over previous
import math

import jax
import jax.numpy as jnp
from jax.experimental import pallas as pl

_T = 16
_K = 4
_C = 256


def _fused_kernel(x_ref, topic_ref, wf_ref, bf_ref, wfast_ref, bfast_ref,
                  wt_ref, bt_ref, wg_ref, bg_ref, out_ref):
    G = topic_ref.shape[0]
    P = jnp.dot(x_ref[...], wf_ref[...], preferred_element_type=jnp.float32)
    P = P + bf_ref[...]
    fast_pre = jnp.dot(topic_ref[...], wfast_ref[...],
                       preferred_element_type=jnp.float32) + bfast_ref[...]
    t = jnp.tanh(jnp.dot(fast_pre, wt_ref[...],
                         preferred_element_type=jnp.float32) + bt_ref[...])
    gate = jax.nn.sigmoid(t)

    P3 = P.reshape(G, _T, _C)
    xm = P3 * gate[:, None, :]

    sim = jax.lax.dot_general(xm, xm, (((2,), (2,)), ((0,), (0,))),
                              preferred_element_type=jnp.float32)
    sim = sim * (1.0 / math.sqrt(_C))

    R = jnp.swapaxes(sim, 0, 1).reshape(_T, G * _T)

    srow = jax.lax.broadcasted_iota(jnp.int32, (_T, G * _T), 0).astype(jnp.float32)
    pw = jnp.exp2(-srow)
    s_work = R
    m0 = None
    for j in range(_K):
        m = jnp.max(s_work, axis=0, keepdims=True)
        if j == 0:
            m0 = m
        q = jnp.where(s_work == m, pw, 0.0)
        mq = jnp.max(q, axis=0, keepdims=True)
        onehot = q == mq
        s_work = jnp.where(onehot, -jnp.inf, s_work)
    e = jnp.where(s_work == -jnp.inf, jnp.exp(R - m0), 0.0)
    denom = jnp.sum(e, axis=0, keepdims=True)
    AB = (e * (1.0 / denom)).reshape(_T, G, _T)

    agg = jax.lax.dot_general(jnp.swapaxes(AB, 0, 1), P3,
                              (((1,), (1,)), ((0,), (0,))),
                              preferred_element_type=jnp.float32)

    out = jnp.dot(agg.reshape(G * _T, _C), wg_ref[...],
                  preferred_element_type=jnp.float32) + bg_ref[...]
    out_ref[...] = jnp.maximum(out + P, 0.0)


def kernel(frame_features, slow_result, fast_result, W_frame, b_frame,
           W_fast, b_fast, Wt, bt, Wg, bg):
    B, L, FD = frame_features.shape
    LF = fast_result.shape[1]
    C = W_frame.shape[1]
    target_len = math.ceil(L / 16) * 16
    x2d = frame_features.reshape(B * L, FD)
    if target_len != L:
        pad = jnp.zeros((B * (target_len - L), FD), dtype=x2d.dtype)
        x2d = jnp.concatenate([x2d, pad], axis=0)
    N = B * LF

    fast2d = fast_result.reshape(N, -1)

    g, cg, _ = Wg.shape
    Wbig = (jnp.eye(g, dtype=Wg.dtype)[:, None, :, None]
            * Wg[:, :, None, :]).reshape(g * cg, g * cg)

    GRAPHS_PER_TILE = 64
    n_tiles = N // GRAPHS_PER_TILE
    rows = GRAPHS_PER_TILE * _T

    out = pl.pallas_call(
        _fused_kernel,
        grid=(n_tiles,),
        in_specs=[
            pl.BlockSpec((rows, FD), lambda i: (i, 0)),
            pl.BlockSpec((GRAPHS_PER_TILE, fast2d.shape[1]), lambda i: (i, 0)),
            pl.BlockSpec((FD, C), lambda i: (0, 0)),
            pl.BlockSpec((1, C), lambda i: (0, 0)),
            pl.BlockSpec((W_fast.shape[0], C), lambda i: (0, 0)),
            pl.BlockSpec((1, C), lambda i: (0, 0)),
            pl.BlockSpec((C, C), lambda i: (0, 0)),
            pl.BlockSpec((1, C), lambda i: (0, 0)),
            pl.BlockSpec((C, C), lambda i: (0, 0)),
            pl.BlockSpec((1, C), lambda i: (0, 0)),
        ],
        out_specs=pl.BlockSpec((rows, C), lambda i: (i, 0)),
        out_shape=jax.ShapeDtypeStruct((N * _T, C), jnp.float32),
    )(x2d, fast2d, W_frame, b_frame.reshape(1, C), W_fast,
      b_fast.reshape(1, C), Wt, bt.reshape(1, C), Wbig, bg.reshape(1, C))

    return out.reshape(B, target_len, C)


# speedup vs baseline: 41.7191x; 1.0384x over previous
import math

import jax
import jax.numpy as jnp
from jax.experimental import pallas as pl

_T = 16
_K = 4
_C = 256


def _fused_kernel(x_ref, topic_ref, wf_ref, bf_ref, wfast_ref, bfast_ref,
                  wt_ref, bt_ref, wg_ref, bg_ref, out_ref):
    G = topic_ref.shape[0]
    P = jnp.dot(x_ref[...], wf_ref[...], preferred_element_type=jnp.float32)
    P = P + bf_ref[...]
    fast_pre = jnp.dot(topic_ref[...], wfast_ref[...],
                       preferred_element_type=jnp.float32) + bfast_ref[...]
    t = jnp.tanh(jnp.dot(fast_pre, wt_ref[...],
                         preferred_element_type=jnp.float32) + bt_ref[...])
    gate = jax.nn.sigmoid(t)

    P3 = P.reshape(G, _T, _C)
    xm = P3 * gate[:, None, :]

    sim = jax.lax.dot_general(xm, xm, (((2,), (2,)), ((0,), (0,))),
                              preferred_element_type=jnp.float32)
    sim = sim * (1.0 / math.sqrt(_C))

    R = jnp.swapaxes(sim, 0, 1).reshape(_T, G * _T)

    srow = jax.lax.broadcasted_iota(jnp.int32, (_T, G * _T), 0).astype(jnp.float32)
    pw = jnp.exp2(-srow)
    s_work = R
    m0 = None
    for j in range(_K):
        m = jnp.max(s_work, axis=0, keepdims=True)
        if j == 0:
            m0 = m
        q = jnp.where(s_work == m, pw, 0.0)
        mq = jnp.max(q, axis=0, keepdims=True)
        onehot = q == mq
        s_work = jnp.where(onehot, -jnp.inf, s_work)
    e = jnp.where(s_work == -jnp.inf, jnp.exp(R - m0), 0.0)
    denom = jnp.sum(e, axis=0, keepdims=True)
    AB = (e * (1.0 / denom)).reshape(_T, G, _T)

    agg = jax.lax.dot_general(jnp.swapaxes(AB, 0, 1), P3,
                              (((1,), (1,)), ((0,), (0,))),
                              preferred_element_type=jnp.float32)

    out = jnp.dot(agg.reshape(G * _T, _C), wg_ref[...],
                  preferred_element_type=jnp.float32) + bg_ref[...]
    out_ref[...] = jnp.maximum(out + P, 0.0)


def kernel(frame_features, slow_result, fast_result, W_frame, b_frame,
           W_fast, b_fast, Wt, bt, Wg, bg):
    B, L, FD = frame_features.shape
    LF = fast_result.shape[1]
    C = W_frame.shape[1]
    target_len = math.ceil(L / 16) * 16
    x2d = frame_features.reshape(B * L, FD)
    if target_len != L:
        pad = jnp.zeros((B * (target_len - L), FD), dtype=x2d.dtype)
        x2d = jnp.concatenate([x2d, pad], axis=0)
    N = B * LF

    fast2d = fast_result.reshape(N, -1)

    g, cg, _ = Wg.shape
    Wbig = (jnp.eye(g, dtype=Wg.dtype)[:, None, :, None]
            * Wg[:, :, None, :]).reshape(g * cg, g * cg)

    GRAPHS_PER_TILE = 128
    n_tiles = N // GRAPHS_PER_TILE
    rows = GRAPHS_PER_TILE * _T

    out = pl.pallas_call(
        _fused_kernel,
        grid=(n_tiles,),
        in_specs=[
            pl.BlockSpec((rows, FD), lambda i: (i, 0)),
            pl.BlockSpec((GRAPHS_PER_TILE, fast2d.shape[1]), lambda i: (i, 0)),
            pl.BlockSpec((FD, C), lambda i: (0, 0)),
            pl.BlockSpec((1, C), lambda i: (0, 0)),
            pl.BlockSpec((W_fast.shape[0], C), lambda i: (0, 0)),
            pl.BlockSpec((1, C), lambda i: (0, 0)),
            pl.BlockSpec((C, C), lambda i: (0, 0)),
            pl.BlockSpec((1, C), lambda i: (0, 0)),
            pl.BlockSpec((C, C), lambda i: (0, 0)),
            pl.BlockSpec((1, C), lambda i: (0, 0)),
        ],
        out_specs=pl.BlockSpec((rows, C), lambda i: (i, 0)),
        out_shape=jax.ShapeDtypeStruct((N * _T, C), jnp.float32),
    )(x2d, fast2d, W_frame, b_frame.reshape(1, C), W_fast,
      b_fast.reshape(1, C), Wt, bt.reshape(1, C), Wbig, bg.reshape(1, C))

    return out.reshape(B, target_len, C)
